# Initial kernel scaffold; baseline (speedup 1.0000x reference)
import jax
import jax.numpy as jnp
from jax.experimental import pallas as pl


def kernel(x, edge_index, W1l, b1l, W1r, W2l, b2l, W2r):
    raise NotImplementedError("write your pallas kernel here")


# speedup vs baseline: 11.5852x; 11.5852x over previous
import functools

import jax
import jax.numpy as jnp
from jax import lax
from jax.experimental import pallas as pl
from jax.experimental.pallas import tpu as pltpu
from jax.experimental.pallas import tpu_sc as plsc

N = 10000
E = 320000
D_IN = 128
HID = 16
D_OUT = 64

NC = 2
NS = 16
NW = NC * NS
CH = 128
NCHUNK = E // CH
CPW = NCHUNK // NW
NTAIL = NCHUNK - NW * CPW
NZT = 10
RPT = N // NZT


def _sc_segment_sum(with_count):
    out_type = [jax.ShapeDtypeStruct((NC, N, HID), jnp.float32)]
    scratch = [
        pltpu.VMEM((CPW, CH), jnp.int32),
        pltpu.VMEM((CPW, CH), jnp.int32),
        pltpu.VMEM((CH,), jnp.int32),
        pltpu.VMEM((CH,), jnp.int32),
        pltpu.VMEM((CH, HID), jnp.float32),
        pltpu.VMEM_SHARED((N, HID), jnp.float32),
    ]
    if with_count:
        out_type.append(jax.ShapeDtypeStruct((NC, N, HID), jnp.float32))
        scratch.append(pltpu.VMEM((CH, HID), jnp.float32))
        scratch.append(pltpu.VMEM_SHARED((N, HID), jnp.float32))

    mesh = plsc.VectorSubcoreMesh(core_axis_name="c", subcore_axis_name="s")

    def body(table, src2, dst2, zeros, *rest):
        if with_count:
            (ones, acc_out, cnt_out,
             src_v, dst_v, tsrc_v, tdst_v, rows_v, acc_sh, ones_v, cnt_sh) = rest
        else:
            (acc_out,
             src_v, dst_v, tsrc_v, tdst_v, rows_v, acc_sh) = rest
        c = lax.axis_index("c")
        s = lax.axis_index("s")
        w = s * NC + c

        @pl.when(s < NZT)
        def _zero():
            pltpu.sync_copy(zeros.at[pl.ds(s * RPT, RPT)],
                            acc_sh.at[pl.ds(s * RPT, RPT)])
            if with_count:
                pltpu.sync_copy(zeros.at[pl.ds(s * RPT, RPT)],
                                cnt_sh.at[pl.ds(s * RPT, RPT)])

        if with_count:
            pltpu.sync_copy(ones, ones_v)
        pltpu.sync_copy(src2.at[pl.ds(w * CPW, CPW)], src_v)
        pltpu.sync_copy(dst2.at[pl.ds(w * CPW, CPW)], dst_v)
        plsc.subcore_barrier()

        def chunk(j, carry):
            pltpu.sync_copy(table.at[src_v.at[j]], rows_v)
            pltpu.sync_copy(rows_v, acc_sh.at[dst_v.at[j]], add=True)
            if with_count:
                pltpu.sync_copy(ones_v, cnt_sh.at[dst_v.at[j]], add=True)
            return carry

        lax.fori_loop(0, CPW, chunk, 0)

        @pl.when(w < NTAIL)
        def _tail():
            r = NW * CPW + w
            pltpu.sync_copy(src2.at[r], tsrc_v)
            pltpu.sync_copy(dst2.at[r], tdst_v)
            pltpu.sync_copy(table.at[tsrc_v], rows_v)
            pltpu.sync_copy(rows_v, acc_sh.at[tdst_v], add=True)
            if with_count:
                pltpu.sync_copy(ones_v, cnt_sh.at[tdst_v], add=True)

        plsc.subcore_barrier()

        @pl.when(s < NZT)
        def _copy_out():
            pltpu.sync_copy(acc_sh.at[pl.ds(s * RPT, RPT)],
                            acc_out.at[c, pl.ds(s * RPT, RPT)])
            if with_count:
                pltpu.sync_copy(cnt_sh.at[pl.ds(s * RPT, RPT)],
                                cnt_out.at[c, pl.ds(s * RPT, RPT)])

    return pl.kernel(body, out_type=tuple(out_type), mesh=mesh,
                     scratch_types=scratch,
                     compiler_params=pltpu.CompilerParams(
                         use_tc_tiling_on_sc=False))


_sc_pass_count = _sc_segment_sum(with_count=True)
_sc_pass = _sc_segment_sum(with_count=False)


def _proj_body(x_ref, w_ref, xp_ref, xr_ref):
    r = jnp.dot(x_ref[...], w_ref[...], preferred_element_type=jnp.float32)
    xp_ref[...] = r[:, :HID]
    xr_ref[...] = r[:, HID:]


def _layer1_body(a0, a1, c0, c1, xr, b, o_ref):
    mean = (a0[...] + a1[...]) / jnp.maximum(c0[...] + c1[...], 1.0)
    o1 = mean + b[...] + xr[...]
    nrm = jnp.sqrt(jnp.sum(o1 * o1, axis=1, keepdims=True))
    o_ref[...] = jnp.maximum(o1 / jnp.maximum(nrm, 1e-12), 0.0)


def _layer2_body(a0, a1, c0, c1, h, w_ref, b, o_ref):
    mean2 = (a0[...] + a1[...]) / jnp.maximum(c0[...] + c1[...], 1.0)
    z = jnp.dot(jnp.concatenate([mean2, h[...]], axis=1), w_ref[...],
                preferred_element_type=jnp.float32) + b[...]
    nrm = jnp.sqrt(jnp.sum(z * z, axis=1, keepdims=True))
    z = z / jnp.maximum(nrm, 1e-12)
    m = jnp.max(z, axis=1, keepdims=True)
    lse = jnp.log(jnp.sum(jnp.exp(z - m), axis=1, keepdims=True)) + m
    o_ref[...] = z - lse


_f32 = jnp.float32


def kernel(x, edge_index, W1l, b1l, W1r, W2l, b2l, W2r):
    src2 = edge_index[0].reshape(NCHUNK, CH)
    dst2 = edge_index[1].reshape(NCHUNK, CH)
    zeros = jnp.zeros((N, HID), _f32)
    ones = jnp.ones((CH, HID), _f32)

    wcat1 = jnp.concatenate([W1l.T, W1r.T], axis=1)
    xp, xr = pl.pallas_call(
        _proj_body,
        out_shape=[jax.ShapeDtypeStruct((N, HID), _f32),
                   jax.ShapeDtypeStruct((N, HID), _f32)],
    )(x, wcat1)

    acc1, cnt = _sc_pass_count(xp, src2, dst2, zeros, ones)

    h = pl.pallas_call(
        _layer1_body,
        out_shape=jax.ShapeDtypeStruct((N, HID), _f32),
    )(acc1[0], acc1[1], cnt[0], cnt[1], xr, b1l.reshape(1, HID))

    acc2, = _sc_pass(h, src2, dst2, zeros)

    wcat2 = jnp.concatenate([W2l.T, W2r.T], axis=0)
    out = pl.pallas_call(
        _layer2_body,
        out_shape=jax.ShapeDtypeStruct((N, D_OUT), _f32),
    )(acc2[0], acc2[1], cnt[0], cnt[1], h, wcat2, b2l.reshape(1, D_OUT))
    return out


# speedup vs baseline: 20.9387x; 1.8074x over previous
import jax
import jax.numpy as jnp
from jax import lax
from jax.experimental import pallas as pl
from jax.experimental.pallas import tpu as pltpu
from jax.experimental.pallas import tpu_sc as plsc

N = 10000
E = 320000
D_IN = 128
HID = 16
D_OUT = 64

NC = 2
NS = 16
NW = NC * NS
CH = 128
NCHUNK = E // CH
CPW = NCHUNK // NW
NTAIL = NCHUNK - NW * CPW
NZT = 10
RPT = N // NZT
K = 6
ROUNDS = CPW // K


def _sc_segment_sum(with_count):
    out_type = [jax.ShapeDtypeStruct((NC, N, HID), jnp.float32)]
    scratch = [
        pltpu.VMEM((CPW, CH), jnp.int32),
        pltpu.VMEM((CPW, CH), jnp.int32),
        pltpu.VMEM((CH,), jnp.int32),
        pltpu.VMEM((CH,), jnp.int32),
        pltpu.VMEM_SHARED((N, HID), jnp.float32),
    ]
    scratch += [pltpu.VMEM((CH, HID), jnp.float32)] * K
    scratch += [pltpu.SemaphoreType.DMA] * (2 * K)
    if with_count:
        out_type.append(jax.ShapeDtypeStruct((NC, N, HID), jnp.float32))
        scratch.append(pltpu.VMEM((CH, HID), jnp.float32))
        scratch.append(pltpu.VMEM_SHARED((N, HID), jnp.float32))
        scratch += [pltpu.SemaphoreType.DMA] * K

    mesh = plsc.VectorSubcoreMesh(core_axis_name="c", subcore_axis_name="s")

    def body(table, src2, dst2, zeros, *rest):
        if with_count:
            (ones, acc_out, cnt_out, src_v, dst_v, tsrc_v, tdst_v,
             acc_sh, *rest2) = rest
            rows = rest2[:K]
            gsem = rest2[K:2 * K]
            ssem = rest2[2 * K:3 * K]
            ones_v = rest2[3 * K]
            cnt_sh = rest2[3 * K + 1]
            csem = rest2[3 * K + 2:]
        else:
            (acc_out, src_v, dst_v, tsrc_v, tdst_v, acc_sh, *rest2) = rest
            rows = rest2[:K]
            gsem = rest2[K:2 * K]
            ssem = rest2[2 * K:3 * K]
        c = lax.axis_index("c")
        s = lax.axis_index("s")
        w = s * NC + c

        @pl.when(s < NZT)
        def _zero():
            pltpu.sync_copy(zeros.at[pl.ds(s * RPT, RPT)],
                            acc_sh.at[pl.ds(s * RPT, RPT)])
            if with_count:
                pltpu.sync_copy(zeros.at[pl.ds(s * RPT, RPT)],
                                cnt_sh.at[pl.ds(s * RPT, RPT)])

        if with_count:
            pltpu.sync_copy(ones, ones_v)
        pltpu.sync_copy(src2.at[pl.ds(w * CPW, CPW)], src_v)
        pltpu.sync_copy(dst2.at[pl.ds(w * CPW, CPW)], dst_v)
        plsc.subcore_barrier()

        def round_body(t, carry):
            for b in range(K):
                j = t * K + b

                @pl.when(t > 0)
                def _drain(b=b, j=j):
                    pltpu.make_async_copy(
                        rows[b], acc_sh.at[dst_v.at[j]], ssem[b]).wait()
                    if with_count:
                        pltpu.make_async_copy(
                            ones_v, cnt_sh.at[dst_v.at[j]], csem[b]).wait()

                pltpu.async_copy(table.at[src_v.at[j]], rows[b], gsem[b])
            for b in range(K):
                j = t * K + b
                pltpu.make_async_copy(
                    table.at[src_v.at[j]], rows[b], gsem[b]).wait()
                pltpu.async_copy(rows[b], acc_sh.at[dst_v.at[j]], ssem[b],
                                 add=True)
                if with_count:
                    pltpu.async_copy(ones_v, cnt_sh.at[dst_v.at[j]], csem[b],
                                     add=True)
            return carry

        lax.fori_loop(0, ROUNDS, round_body, 0)
        for b in range(K):
            pltpu.make_async_copy(rows[b], acc_sh.at[dst_v.at[b]],
                                  ssem[b]).wait()
            if with_count:
                pltpu.make_async_copy(ones_v, cnt_sh.at[dst_v.at[b]],
                                      csem[b]).wait()

        @pl.when(w < NTAIL)
        def _tail():
            r = NW * CPW + w
            pltpu.sync_copy(src2.at[r], tsrc_v)
            pltpu.sync_copy(dst2.at[r], tdst_v)
            pltpu.sync_copy(table.at[tsrc_v], rows[0])
            pltpu.sync_copy(rows[0], acc_sh.at[tdst_v], add=True)
            if with_count:
                pltpu.sync_copy(ones_v, cnt_sh.at[tdst_v], add=True)

        plsc.subcore_barrier()

        @pl.when(s < NZT)
        def _copy_out():
            pltpu.sync_copy(acc_sh.at[pl.ds(s * RPT, RPT)],
                            acc_out.at[c, pl.ds(s * RPT, RPT)])
            if with_count:
                pltpu.sync_copy(cnt_sh.at[pl.ds(s * RPT, RPT)],
                                cnt_out.at[c, pl.ds(s * RPT, RPT)])

    return pl.kernel(body, out_type=tuple(out_type), mesh=mesh,
                     scratch_types=scratch,
                     compiler_params=pltpu.CompilerParams(
                         use_tc_tiling_on_sc=False))


_sc_pass_count = _sc_segment_sum(with_count=True)
_sc_pass = _sc_segment_sum(with_count=False)


def _proj_body(x_ref, w_ref, xp_ref, xr_ref):
    r = jnp.dot(x_ref[...], w_ref[...], preferred_element_type=jnp.float32)
    xp_ref[...] = r[:, :HID]
    xr_ref[...] = r[:, HID:]


def _layer1_body(acc, cnt, xr, b, o_ref):
    mean = (acc[0] + acc[1]) / jnp.maximum(cnt[0] + cnt[1], 1.0)
    o1 = mean + b[...] + xr[...]
    nrm = jnp.sqrt(jnp.sum(o1 * o1, axis=1, keepdims=True))
    o_ref[...] = jnp.maximum(o1 / jnp.maximum(nrm, 1e-12), 0.0)


def _layer2_body(acc, cnt, h, w_ref, b, o_ref):
    mean2 = (acc[0] + acc[1]) / jnp.maximum(cnt[0] + cnt[1], 1.0)
    z = jnp.dot(jnp.concatenate([mean2, h[...]], axis=1), w_ref[...],
                preferred_element_type=jnp.float32) + b[...]
    nrm = jnp.sqrt(jnp.sum(z * z, axis=1, keepdims=True))
    z = z / jnp.maximum(nrm, 1e-12)
    m = jnp.max(z, axis=1, keepdims=True)
    lse = jnp.log(jnp.sum(jnp.exp(z - m), axis=1, keepdims=True)) + m
    o_ref[...] = z - lse


_f32 = jnp.float32


def kernel(x, edge_index, W1l, b1l, W1r, W2l, b2l, W2r):
    src2 = edge_index[0].reshape(NCHUNK, CH)
    dst2 = edge_index[1].reshape(NCHUNK, CH)
    zeros = jnp.zeros((N, HID), _f32)
    ones = jnp.ones((CH, HID), _f32)

    wcat1 = jnp.concatenate([W1l.T, W1r.T], axis=1)
    xp, xr = pl.pallas_call(
        _proj_body,
        out_shape=[jax.ShapeDtypeStruct((N, HID), _f32),
                   jax.ShapeDtypeStruct((N, HID), _f32)],
    )(x, wcat1)

    acc1, cnt = _sc_pass_count(xp, src2, dst2, zeros, ones)

    h = pl.pallas_call(
        _layer1_body,
        out_shape=jax.ShapeDtypeStruct((N, HID), _f32),
    )(acc1, cnt, xr, b1l.reshape(1, HID))

    acc2, = _sc_pass(h, src2, dst2, zeros)

    wcat2 = jnp.concatenate([W2l.T, W2r.T], axis=0)
    out = pl.pallas_call(
        _layer2_body,
        out_shape=jax.ShapeDtypeStruct((N, D_OUT), _f32),
    )(acc2, cnt, h, wcat2, b2l.reshape(1, D_OUT))
    return out


# speedup vs baseline: 29.0812x; 1.3889x over previous
import jax
import jax.numpy as jnp
from jax import lax
from jax.experimental import pallas as pl
from jax.experimental.pallas import tpu as pltpu
from jax.experimental.pallas import tpu_sc as plsc

N = 10000
NP = 10240
E = 320000
D_IN = 128
HID = 16
D_OUT = 64
PR = NP // 8

NC = 2
NS = 16
NW = NC * NS
CH = 128
NCHUNK = E // CH
CPW = NCHUNK // NW
NTAIL = NCHUNK - NW * CPW
RPT = NP // NS
K = 6
ROUNDS = CPW // K


def _sc_segment_sum(with_count):
    out_type = [jax.ShapeDtypeStruct((NC, NP, HID), jnp.float32)]
    scratch = [
        pltpu.VMEM((CPW, CH), jnp.int32),
        pltpu.VMEM((CPW, CH), jnp.int32),
        pltpu.VMEM((CH,), jnp.int32),
        pltpu.VMEM((CH,), jnp.int32),
        pltpu.VMEM_SHARED((NP, HID), jnp.float32),
    ]
    scratch += [pltpu.VMEM((CH, HID), jnp.float32)] * K
    scratch += [pltpu.SemaphoreType.DMA] * (2 * K)
    if with_count:
        out_type.append(jax.ShapeDtypeStruct((NC, NP, HID), jnp.float32))
        scratch.append(pltpu.VMEM((CH, HID), jnp.float32))
        scratch.append(pltpu.VMEM_SHARED((NP, HID), jnp.float32))
        scratch += [pltpu.SemaphoreType.DMA] * K

    mesh = plsc.VectorSubcoreMesh(core_axis_name="c", subcore_axis_name="s")

    def body(table, ei3, zeros, *rest):
        if with_count:
            (ones, acc_out, cnt_out, src_v, dst_v, tsrc_v, tdst_v,
             acc_sh, *rest2) = rest
            rows = rest2[:K]
            gsem = rest2[K:2 * K]
            ssem = rest2[2 * K:3 * K]
            ones_v = rest2[3 * K]
            cnt_sh = rest2[3 * K + 1]
            csem = rest2[3 * K + 2:]
        else:
            (acc_out, src_v, dst_v, tsrc_v, tdst_v, acc_sh, *rest2) = rest
            rows = rest2[:K]
            gsem = rest2[K:2 * K]
            ssem = rest2[2 * K:3 * K]
        c = lax.axis_index("c")
        s = lax.axis_index("s")
        w = s * NC + c

        pltpu.sync_copy(zeros.at[pl.ds(s * RPT, RPT)],
                        acc_sh.at[pl.ds(s * RPT, RPT)])
        if with_count:
            pltpu.sync_copy(zeros.at[pl.ds(s * RPT, RPT)],
                            cnt_sh.at[pl.ds(s * RPT, RPT)])
            pltpu.sync_copy(ones, ones_v)
        pltpu.sync_copy(ei3.at[0, pl.ds(w * CPW, CPW)], src_v)
        pltpu.sync_copy(ei3.at[1, pl.ds(w * CPW, CPW)], dst_v)
        plsc.subcore_barrier()

        def round_body(t, carry):
            for b in range(K):
                j = t * K + b

                @pl.when(t > 0)
                def _drain(b=b, j=j):
                    pltpu.make_async_copy(
                        rows[b], acc_sh.at[dst_v.at[j]], ssem[b]).wait()
                    if with_count:
                        pltpu.make_async_copy(
                            ones_v, cnt_sh.at[dst_v.at[j]], csem[b]).wait()

                pltpu.async_copy(table.at[src_v.at[j]], rows[b], gsem[b])
            for b in range(K):
                j = t * K + b
                pltpu.make_async_copy(
                    table.at[src_v.at[j]], rows[b], gsem[b]).wait()
                pltpu.async_copy(rows[b], acc_sh.at[dst_v.at[j]], ssem[b],
                                 add=True)
                if with_count:
                    pltpu.async_copy(ones_v, cnt_sh.at[dst_v.at[j]], csem[b],
                                     add=True)
            return carry

        lax.fori_loop(0, ROUNDS, round_body, 0)
        for b in range(K):
            pltpu.make_async_copy(rows[b], acc_sh.at[dst_v.at[b]],
                                  ssem[b]).wait()
            if with_count:
                pltpu.make_async_copy(ones_v, cnt_sh.at[dst_v.at[b]],
                                      csem[b]).wait()

        @pl.when(w < NTAIL)
        def _tail():
            r = NW * CPW + w
            pltpu.sync_copy(ei3.at[0, r], tsrc_v)
            pltpu.sync_copy(ei3.at[1, r], tdst_v)
            pltpu.sync_copy(table.at[tsrc_v], rows[0])
            pltpu.sync_copy(rows[0], acc_sh.at[tdst_v], add=True)
            if with_count:
                pltpu.sync_copy(ones_v, cnt_sh.at[tdst_v], add=True)

        plsc.subcore_barrier()
        pltpu.sync_copy(acc_sh.at[pl.ds(s * RPT, RPT)],
                        acc_out.at[c, pl.ds(s * RPT, RPT)])
        if with_count:
            pltpu.sync_copy(cnt_sh.at[pl.ds(s * RPT, RPT)],
                            cnt_out.at[c, pl.ds(s * RPT, RPT)])

    return pl.kernel(body, out_type=tuple(out_type), mesh=mesh,
                     scratch_types=scratch,
                     compiler_params=pltpu.CompilerParams(
                         use_tc_tiling_on_sc=False))


_sc_pass_count = _sc_segment_sum(with_count=True)
_sc_pass = _sc_segment_sum(with_count=False)


def _proj_body(x_ref, w_ref, xp_ref, xr_ref):
    r = jnp.dot(x_ref[...], w_ref[...], preferred_element_type=jnp.float32)
    xp_ref[...] = r[:, :HID]
    xr_ref[...] = r[:, HID:]


def _seg_mask():
    i = lax.broadcasted_iota(jnp.int32, (128, 128), 0) // HID
    j = lax.broadcasted_iota(jnp.int32, (128, 128), 1) // HID
    return (i == j).astype(jnp.float32)


def _layer1_body(acc, cnt, xr, b, o_ref):
    mean = (acc[0] + acc[1]) / jnp.maximum(cnt[0] + cnt[1], 1.0)
    o1 = mean + jnp.tile(b[...], (1, 8)) + xr[...]
    nrm2 = jnp.dot(o1 * o1, _seg_mask(), preferred_element_type=jnp.float32)
    o_ref[...] = jnp.maximum(o1 / jnp.maximum(jnp.sqrt(nrm2), 1e-12), 0.0)


def _layer2_body(acc, cnt, h, w_ref, b, o_ref):
    mean2 = (acc[0] + acc[1]) / jnp.maximum(cnt[0] + cnt[1], 1.0)
    hh = h[...]
    for m in range(8):
        cm = jnp.concatenate([mean2[:, HID * m:HID * (m + 1)],
                              hh[:, HID * m:HID * (m + 1)]], axis=1)
        z = jnp.dot(cm, w_ref[...], preferred_element_type=jnp.float32) + b[...]
        nrm = jnp.sqrt(jnp.sum(z * z, axis=1, keepdims=True))
        z = z / jnp.maximum(nrm, 1e-12)
        lse = jnp.log(jnp.sum(jnp.exp(z - 1.0), axis=1, keepdims=True)) + 1.0
        o_ref[m] = z - lse


_f32 = jnp.float32


def kernel(x, edge_index, W1l, b1l, W1r, W2l, b2l, W2r):
    ei3 = edge_index.reshape(2, NCHUNK, CH)
    zeros = jnp.zeros((NP, HID), _f32)
    ones = jnp.ones((CH, HID), _f32)

    wcat1 = jnp.concatenate([W1l.T, W1r.T], axis=1)
    xp, xr = pl.pallas_call(
        _proj_body,
        out_shape=[jax.ShapeDtypeStruct((N, HID), _f32),
                   jax.ShapeDtypeStruct((N, HID), _f32)],
    )(x, wcat1)
    xr_p = jnp.concatenate([xr, jnp.zeros((NP - N, HID), _f32)]
                           ).reshape(PR, 128)

    acc1, cnt = _sc_pass_count(xp, ei3, zeros, ones)
    accp = acc1.reshape(NC, PR, 128)
    cntp = cnt.reshape(NC, PR, 128)

    hp = pl.pallas_call(
        _layer1_body,
        out_shape=jax.ShapeDtypeStruct((PR, 128), _f32),
    )(accp, cntp, xr_p, b1l.reshape(1, HID))

    h_lin = hp.reshape(NP, HID)

    acc2, = _sc_pass(h_lin, ei3, zeros)
    acc2p = acc2.reshape(NC, PR, 128)

    wcat2 = jnp.concatenate([W2l.T, W2r.T], axis=0)
    stacked = pl.pallas_call(
        _layer2_body,
        out_shape=jax.ShapeDtypeStruct((8, PR, D_OUT), _f32),
    )(acc2p, cntp, hp, wcat2, b2l.reshape(1, D_OUT))
    return stacked.transpose(1, 0, 2).reshape(NP, D_OUT)[:N]


# speedup vs baseline: 30.7427x; 1.0571x over previous
import jax
import jax.numpy as jnp
from jax import lax
from jax.experimental import pallas as pl
from jax.experimental.pallas import tpu as pltpu
from jax.experimental.pallas import tpu_sc as plsc

N = 10000
NP = 10240
E = 320000
D_IN = 128
HID = 16
D_OUT = 64
PR = NP // 8

NC = 2
NS = 16
NW = NC * NS
CH = 128
NCHUNK = E // CH
CPW = NCHUNK // NW
NTAIL = NCHUNK - NW * CPW
RPT = NP // NS
K = 6
ROUNDS = CPW // K


def _sc_segment_sum(with_count):
    out_type = [jax.ShapeDtypeStruct((NC, NP, HID), jnp.float32)]
    scratch = [
        pltpu.VMEM((CPW, CH), jnp.int32),
        pltpu.VMEM((CPW, CH), jnp.int32),
        pltpu.VMEM((CH,), jnp.int32),
        pltpu.VMEM((CH,), jnp.int32),
        pltpu.VMEM_SHARED((NP, HID), jnp.float32),
    ]
    scratch += [pltpu.VMEM((CH, HID), jnp.float32)] * K
    scratch += [pltpu.SemaphoreType.DMA] * (2 * K)
    scratch.append(pltpu.VMEM((CH, HID), jnp.float32))
    if with_count:
        out_type.append(jax.ShapeDtypeStruct((NC, NP, HID), jnp.float32))
        scratch.append(pltpu.VMEM((CH, HID), jnp.float32))
        scratch.append(pltpu.VMEM_SHARED((NP, HID), jnp.float32))
        scratch += [pltpu.SemaphoreType.DMA] * K

    mesh = plsc.VectorSubcoreMesh(core_axis_name="c", subcore_axis_name="s")

    def body(table, ei3, *rest):
        if with_count:
            (acc_out, cnt_out, src_v, dst_v, tsrc_v, tdst_v,
             acc_sh, *rest2) = rest
            rows = rest2[:K]
            gsem = rest2[K:2 * K]
            ssem = rest2[2 * K:3 * K]
            zero_v = rest2[3 * K]
            ones_v = rest2[3 * K + 1]
            cnt_sh = rest2[3 * K + 2]
            csem = rest2[3 * K + 3:]
        else:
            (acc_out, src_v, dst_v, tsrc_v, tdst_v, acc_sh, *rest2) = rest
            rows = rest2[:K]
            gsem = rest2[K:2 * K]
            ssem = rest2[2 * K:3 * K]
            zero_v = rest2[3 * K]
        c = lax.axis_index("c")
        s = lax.axis_index("s")
        w = s * NC + c

        def fill(i, _):
            zero_v[i] = jnp.zeros((HID,), jnp.float32)
            if with_count:
                ones_v[i] = jnp.ones((HID,), jnp.float32)
            return 0
        lax.fori_loop(0, CH, fill, 0)
        for k in range(RPT // CH):
            pltpu.sync_copy(zero_v, acc_sh.at[pl.ds(s * RPT + k * CH, CH)])
            if with_count:
                pltpu.sync_copy(zero_v,
                                cnt_sh.at[pl.ds(s * RPT + k * CH, CH)])
        pltpu.sync_copy(ei3.at[0, pl.ds(w * CPW, CPW)], src_v)
        pltpu.sync_copy(ei3.at[1, pl.ds(w * CPW, CPW)], dst_v)
        plsc.subcore_barrier()

        def round_body(t, carry):
            for b in range(K):
                j = t * K + b

                @pl.when(t > 0)
                def _drain(b=b, j=j):
                    pltpu.make_async_copy(
                        rows[b], acc_sh.at[dst_v.at[j]], ssem[b]).wait()
                    if with_count:
                        pltpu.make_async_copy(
                            ones_v, cnt_sh.at[dst_v.at[j]], csem[b]).wait()

                pltpu.async_copy(table.at[src_v.at[j]], rows[b], gsem[b])
            for b in range(K):
                j = t * K + b
                pltpu.make_async_copy(
                    table.at[src_v.at[j]], rows[b], gsem[b]).wait()
                pltpu.async_copy(rows[b], acc_sh.at[dst_v.at[j]], ssem[b],
                                 add=True)
                if with_count:
                    pltpu.async_copy(ones_v, cnt_sh.at[dst_v.at[j]], csem[b],
                                     add=True)
            return carry

        lax.fori_loop(0, ROUNDS, round_body, 0)
        for b in range(K):
            pltpu.make_async_copy(rows[b], acc_sh.at[dst_v.at[b]],
                                  ssem[b]).wait()
            if with_count:
                pltpu.make_async_copy(ones_v, cnt_sh.at[dst_v.at[b]],
                                      csem[b]).wait()

        @pl.when(w < NTAIL)
        def _tail():
            r = NW * CPW + w
            pltpu.sync_copy(ei3.at[0, r], tsrc_v)
            pltpu.sync_copy(ei3.at[1, r], tdst_v)
            pltpu.sync_copy(table.at[tsrc_v], rows[0])
            pltpu.sync_copy(rows[0], acc_sh.at[tdst_v], add=True)
            if with_count:
                pltpu.sync_copy(ones_v, cnt_sh.at[tdst_v], add=True)

        plsc.subcore_barrier()
        pltpu.sync_copy(acc_sh.at[pl.ds(s * RPT, RPT)],
                        acc_out.at[c, pl.ds(s * RPT, RPT)])
        if with_count:
            pltpu.sync_copy(cnt_sh.at[pl.ds(s * RPT, RPT)],
                            cnt_out.at[c, pl.ds(s * RPT, RPT)])

    return pl.kernel(body, out_type=tuple(out_type), mesh=mesh,
                     scratch_types=scratch,
                     compiler_params=pltpu.CompilerParams(
                         use_tc_tiling_on_sc=False))


_sc_pass_count = _sc_segment_sum(with_count=True)
_sc_pass = _sc_segment_sum(with_count=False)


def _proj_body(x_ref, w_ref, xp_ref, xr_ref):
    r = jnp.dot(x_ref[...], w_ref[...], preferred_element_type=jnp.float32)
    nlast = N - 7 * PR
    for m in range(8):
        lo = m * PR
        if m < 7:
            xp_ref[:, HID * m:HID * (m + 1)] = r[lo:lo + PR, :HID]
            xr_ref[:, HID * m:HID * (m + 1)] = r[lo:lo + PR, HID:]
        else:
            z = jnp.zeros((PR - nlast, HID), jnp.float32)
            xp_ref[:, HID * m:HID * (m + 1)] = jnp.concatenate(
                [r[lo:, :HID], z], axis=0)
            xr_ref[:, HID * m:HID * (m + 1)] = jnp.concatenate(
                [r[lo:, HID:], z], axis=0)


def _seg_mask():
    i = lax.broadcasted_iota(jnp.int32, (128, 128), 0) // HID
    j = lax.broadcasted_iota(jnp.int32, (128, 128), 1) // HID
    return (i == j).astype(jnp.float32)


def _layer1_body(acc, cnt, xr, b, o_ref):
    mean = (acc[0] + acc[1]) / jnp.maximum(cnt[0] + cnt[1], 1.0)
    o1 = mean + jnp.tile(b[...], (1, 8)) + xr[...]
    nrm2 = jnp.dot(o1 * o1, _seg_mask(), preferred_element_type=jnp.float32)
    o_ref[...] = jnp.maximum(o1 / jnp.maximum(jnp.sqrt(nrm2), 1e-12), 0.0)


def _layer2_body(acc, cnt, h, w_ref, b, o_ref):
    mean2 = (acc[0] + acc[1]) / jnp.maximum(cnt[0] + cnt[1], 1.0)
    hh = h[...]
    ones64 = jnp.ones((D_OUT, D_OUT), jnp.float32)
    for m in range(8):
        cm = jnp.concatenate([mean2[:, HID * m:HID * (m + 1)],
                              hh[:, HID * m:HID * (m + 1)]], axis=1)
        z = jnp.dot(cm, w_ref[...], preferred_element_type=jnp.float32) + b[...]
        nrm2 = jnp.dot(z * z, ones64, preferred_element_type=jnp.float32)
        z = z / jnp.maximum(jnp.sqrt(nrm2), 1e-12)
        se = jnp.dot(jnp.exp(z - 1.0), ones64,
                     preferred_element_type=jnp.float32)
        o_ref[m] = z - (jnp.log(se) + 1.0)


_f32 = jnp.float32


def kernel(x, edge_index, W1l, b1l, W1r, W2l, b2l, W2r):
    ei = 8 * (edge_index % PR) + edge_index // PR
    ei3 = ei.reshape(2, NCHUNK, CH)

    wcat1 = jnp.concatenate([W1l.T, W1r.T], axis=1)
    xpp, xrp = pl.pallas_call(
        _proj_body,
        out_shape=[jax.ShapeDtypeStruct((PR, 128), _f32),
                   jax.ShapeDtypeStruct((PR, 128), _f32)],
    )(x, wcat1)
    xp_lin = xpp.reshape(NP, HID)

    acc1, cnt = _sc_pass_count(xp_lin, ei3)
    accp = acc1.reshape(NC, PR, 128)
    cntp = cnt.reshape(NC, PR, 128)

    hp = pl.pallas_call(
        _layer1_body,
        out_shape=jax.ShapeDtypeStruct((PR, 128), _f32),
    )(accp, cntp, xrp, b1l.reshape(1, HID))

    h_lin = hp.reshape(NP, HID)

    acc2, = _sc_pass(h_lin, ei3)
    acc2p = acc2.reshape(NC, PR, 128)

    wcat2 = jnp.concatenate([W2l.T, W2r.T], axis=0)
    stacked = pl.pallas_call(
        _layer2_body,
        out_shape=jax.ShapeDtypeStruct((8, PR, D_OUT), _f32),
    )(acc2p, cntp, hp, wcat2, b2l.reshape(1, D_OUT))
    return stacked.reshape(NP, D_OUT)[:N]


# speedup vs baseline: 33.1167x; 1.0772x over previous
import jax
import jax.numpy as jnp
from jax import lax
from jax.experimental import pallas as pl
from jax.experimental.pallas import tpu as pltpu
from jax.experimental.pallas import tpu_sc as plsc

N = 10000
NP = 10240
E = 320000
D_IN = 128
HID = 16
D_OUT = 64
PR = NP // 8

NC = 2
NS = 16
NW = NC * NS
CH = 128
NCHUNK = E // CH
CPW = NCHUNK // NW
NTAIL = NCHUNK - NW * CPW
RPT = NP // NS
K = 6
ROUNDS = CPW // K


def _sc_segment_sum(with_count):
    out_type = [jax.ShapeDtypeStruct((NC, NP, HID), jnp.float32)]
    scratch = [
        pltpu.VMEM((CPW, CH), jnp.int32),
        pltpu.VMEM((CPW, CH), jnp.int32),
        pltpu.VMEM((CH,), jnp.int32),
        pltpu.VMEM((CH,), jnp.int32),
        pltpu.VMEM_SHARED((NP, HID), jnp.float32),
    ]
    scratch += [pltpu.VMEM((CH, HID), jnp.float32)] * K
    scratch += [pltpu.SemaphoreType.DMA] * (2 * K)
    scratch.append(pltpu.VMEM((CH, HID), jnp.float32))
    if with_count:
        out_type.append(jax.ShapeDtypeStruct((NC, NP, HID), jnp.float32))
        scratch.append(pltpu.VMEM((CH, HID), jnp.float32))
        scratch.append(pltpu.VMEM_SHARED((NP, HID), jnp.float32))
        scratch += [pltpu.SemaphoreType.DMA] * K

    mesh = plsc.VectorSubcoreMesh(core_axis_name="c", subcore_axis_name="s")

    def body(table, ei3, *rest):
        if with_count:
            (acc_out, cnt_out, src_v, dst_v, tsrc_v, tdst_v,
             acc_sh, *rest2) = rest
            rows = rest2[:K]
            gsem = rest2[K:2 * K]
            ssem = rest2[2 * K:3 * K]
            zero_v = rest2[3 * K]
            ones_v = rest2[3 * K + 1]
            cnt_sh = rest2[3 * K + 2]
            csem = rest2[3 * K + 3:]
        else:
            (acc_out, src_v, dst_v, tsrc_v, tdst_v, acc_sh, *rest2) = rest
            rows = rest2[:K]
            gsem = rest2[K:2 * K]
            ssem = rest2[2 * K:3 * K]
            zero_v = rest2[3 * K]
        c = lax.axis_index("c")
        s = lax.axis_index("s")
        w = s * NC + c

        def fill(i, _):
            zero_v[i] = jnp.zeros((HID,), jnp.float32)
            if with_count:
                ones_v[i] = jnp.ones((HID,), jnp.float32)
            return 0
        lax.fori_loop(0, CH, fill, 0)
        for k in range(RPT // CH):
            pltpu.sync_copy(zero_v, acc_sh.at[pl.ds(s * RPT + k * CH, CH)])
            if with_count:
                pltpu.sync_copy(zero_v,
                                cnt_sh.at[pl.ds(s * RPT + k * CH, CH)])
        pltpu.sync_copy(ei3.at[0, pl.ds(w * CPW, CPW)], src_v)
        pltpu.sync_copy(ei3.at[1, pl.ds(w * CPW, CPW)], dst_v)
        plsc.subcore_barrier()

        def round_body(t, carry):
            for b in range(K):
                j = t * K + b

                @pl.when(t > 0)
                def _drain(b=b, j=j):
                    pltpu.make_async_copy(
                        rows[b], acc_sh.at[dst_v.at[j]], ssem[b]).wait()
                    if with_count:
                        pltpu.make_async_copy(
                            ones_v, cnt_sh.at[dst_v.at[j]], csem[b]).wait()

                pltpu.async_copy(table.at[src_v.at[j]], rows[b], gsem[b])
            for b in range(K):
                j = t * K + b
                pltpu.make_async_copy(
                    table.at[src_v.at[j]], rows[b], gsem[b]).wait()
                pltpu.async_copy(rows[b], acc_sh.at[dst_v.at[j]], ssem[b],
                                 add=True)
                if with_count:
                    pltpu.async_copy(ones_v, cnt_sh.at[dst_v.at[j]], csem[b],
                                     add=True)
            return carry

        lax.fori_loop(0, ROUNDS, round_body, 0)
        for b in range(K):
            pltpu.make_async_copy(rows[b], acc_sh.at[dst_v.at[b]],
                                  ssem[b]).wait()
            if with_count:
                pltpu.make_async_copy(ones_v, cnt_sh.at[dst_v.at[b]],
                                      csem[b]).wait()

        @pl.when(w < NTAIL)
        def _tail():
            r = NW * CPW + w
            pltpu.sync_copy(ei3.at[0, r], tsrc_v)
            pltpu.sync_copy(ei3.at[1, r], tdst_v)
            pltpu.sync_copy(table.at[tsrc_v], rows[0])
            pltpu.sync_copy(rows[0], acc_sh.at[tdst_v], add=True)
            if with_count:
                pltpu.sync_copy(ones_v, cnt_sh.at[tdst_v], add=True)

        plsc.subcore_barrier()
        pltpu.sync_copy(acc_sh.at[pl.ds(s * RPT, RPT)],
                        acc_out.at[c, pl.ds(s * RPT, RPT)])
        if with_count:
            pltpu.sync_copy(cnt_sh.at[pl.ds(s * RPT, RPT)],
                            cnt_out.at[c, pl.ds(s * RPT, RPT)])

    return pl.kernel(body, out_type=tuple(out_type), mesh=mesh,
                     scratch_types=scratch,
                     compiler_params=pltpu.CompilerParams(
                         use_tc_tiling_on_sc=False))


_sc_pass_count = _sc_segment_sum(with_count=True)
_sc_pass = _sc_segment_sum(with_count=False)


def _proj_body(x_ref, w_ref, ei_ref, xp_ref, xr_ref, eo_ref):
    e = ei_ref[...]
    q = jnp.right_shift(e * 6554, 23)
    eo_ref[...] = 8 * (e - q * PR) + q
    r = jnp.dot(x_ref[...], w_ref[...], preferred_element_type=jnp.float32)
    nlast = N - 7 * PR
    for m in range(8):
        lo = m * PR
        if m < 7:
            xp_ref[:, HID * m:HID * (m + 1)] = r[lo:lo + PR, :HID]
            xr_ref[:, HID * m:HID * (m + 1)] = r[lo:lo + PR, HID:]
        else:
            z = jnp.zeros((PR - nlast, HID), jnp.float32)
            xp_ref[:, HID * m:HID * (m + 1)] = jnp.concatenate(
                [r[lo:, :HID], z], axis=0)
            xr_ref[:, HID * m:HID * (m + 1)] = jnp.concatenate(
                [r[lo:, HID:], z], axis=0)


def _seg_mask():
    i = lax.broadcasted_iota(jnp.int32, (128, 128), 0) // HID
    j = lax.broadcasted_iota(jnp.int32, (128, 128), 1) // HID
    return (i == j).astype(jnp.float32)


def _layer1_body(acc, cnt, xr, b, o_ref):
    mean = (acc[0] + acc[1]) / jnp.maximum(cnt[0] + cnt[1], 1.0)
    o1 = mean + jnp.tile(b[...], (1, 8)) + xr[...]
    nrm2 = jnp.dot(o1 * o1, _seg_mask(), preferred_element_type=jnp.float32)
    o_ref[...] = jnp.maximum(o1 / jnp.maximum(jnp.sqrt(nrm2), 1e-12), 0.0)


def _layer2_body(acc, cnt, h, w_ref, b, o_ref):
    mean2 = (acc[0] + acc[1]) / jnp.maximum(cnt[0] + cnt[1], 1.0)
    hh = h[...]
    ones64 = jnp.ones((D_OUT, D_OUT), jnp.float32)
    for m in range(8):
        cm = jnp.concatenate([mean2[:, HID * m:HID * (m + 1)],
                              hh[:, HID * m:HID * (m + 1)]], axis=1)
        z = jnp.dot(cm, w_ref[...], preferred_element_type=jnp.float32) + b[...]
        nrm2 = jnp.dot(z * z, ones64, preferred_element_type=jnp.float32)
        z = z / jnp.maximum(jnp.sqrt(nrm2), 1e-12)
        se = jnp.dot(jnp.exp(z - 1.0), ones64,
                     preferred_element_type=jnp.float32)
        out = z - (jnp.log(se) + 1.0)
        if m < 7:
            o_ref[m * PR:(m + 1) * PR] = out
        else:
            o_ref[7 * PR:N] = out[:N - 7 * PR]


_f32 = jnp.float32


def kernel(x, edge_index, W1l, b1l, W1r, W2l, b2l, W2r):
    wcat1 = jnp.concatenate([W1l.T, W1r.T], axis=1)
    xpp, xrp, eo = pl.pallas_call(
        _proj_body,
        out_shape=[jax.ShapeDtypeStruct((PR, 128), _f32),
                   jax.ShapeDtypeStruct((PR, 128), _f32),
                   jax.ShapeDtypeStruct((2 * E // 128, 128), jnp.int32)],
    )(x, wcat1, edge_index.reshape(2 * E // 128, 128))
    xp_lin = xpp.reshape(NP, HID)
    ei3 = eo.reshape(2, NCHUNK, CH)

    acc1, cnt = _sc_pass_count(xp_lin, ei3)
    accp = acc1.reshape(NC, PR, 128)
    cntp = cnt.reshape(NC, PR, 128)

    hp = pl.pallas_call(
        _layer1_body,
        out_shape=jax.ShapeDtypeStruct((PR, 128), _f32),
    )(accp, cntp, xrp, b1l.reshape(1, HID))

    h_lin = hp.reshape(NP, HID)

    acc2, = _sc_pass(h_lin, ei3)
    acc2p = acc2.reshape(NC, PR, 128)

    wcat2 = jnp.concatenate([W2l.T, W2r.T], axis=0)
    return pl.pallas_call(
        _layer2_body,
        out_shape=jax.ShapeDtypeStruct((N, D_OUT), _f32),
    )(acc2p, cntp, hp, wcat2, b2l.reshape(1, D_OUT))
